# Initial kernel scaffold; baseline (speedup 1.0000x reference)
import functools

import jax
import jax.numpy as jnp
from jax import lax
from jax.experimental import pallas as pl
from jax.experimental.pallas import tpu as pltpu
from jax.experimental.pallas import tpu_sc as plsc

_NC = 2
_NS = 16
_NW = _NC * _NS
_CHUNK = 80
_ZR = 125


def _segment_sum_sc(h, src, dst):
    n, d = h.shape
    e = src.shape[0]
    per_w = e // _NW
    n_chunks = per_w // _CHUNK
    rows_per_sub = n // _NS
    mesh = plsc.VectorSubcoreMesh(core_axis_name="c", subcore_axis_name="s")

    @functools.partial(
        pl.kernel,
        out_type=jax.ShapeDtypeStruct((_NC, n, d), jnp.float32),
        mesh=mesh,
        scratch_types=[
            pltpu.VMEM((_CHUNK,), jnp.int32),
            pltpu.VMEM((_CHUNK,), jnp.int32),
            pltpu.VMEM((_CHUNK, d), jnp.float32),
            pltpu.VMEM((_ZR, d), jnp.float32),
            pltpu.VMEM_SHARED((n, d), jnp.float32),
            pltpu.SemaphoreType.DMA,
        ],
    )
    def seg_kernel(h_hbm, src_hbm, dst_hbm, out_hbm,
                   src_v, dst_v, rows_v, zeros_v, acc, sem):
        cid = lax.axis_index("c")
        sid = lax.axis_index("s")
        wid = sid * _NC + cid

        zvec = jnp.zeros((16,), jnp.float32)

        @pl.loop(0, _ZR)
        def _(r):
            @pl.loop(0, d // 16)
            def _(c):
                zeros_v.at[r, pl.ds(c * 16, 16)][...] = zvec

        @pl.loop(0, rows_per_sub // _ZR)
        def _(j):
            pltpu.sync_copy(
                zeros_v, acc.at[pl.ds(sid * rows_per_sub + j * _ZR, _ZR)])

        plsc.subcore_barrier()

        base = wid * per_w

        @pl.loop(0, n_chunks)
        def _(i):
            off = base + i * _CHUNK
            pltpu.sync_copy(src_hbm.at[pl.ds(off, _CHUNK)], src_v)
            pltpu.sync_copy(dst_hbm.at[pl.ds(off, _CHUNK)], dst_v)
            pltpu.async_copy(h_hbm.at[src_v], rows_v, sem).wait()
            pltpu.sync_copy(rows_v, acc.at[dst_v], add=True)

        plsc.subcore_barrier()

        pltpu.sync_copy(
            acc.at[pl.ds(sid * rows_per_sub, rows_per_sub)],
            out_hbm.at[cid, pl.ds(sid * rows_per_sub, rows_per_sub)])

    return seg_kernel(h, src, dst)


def _tc_matmul(x, w):
    def body(x_ref, w_ref, o_ref):
        o_ref[...] = jnp.dot(x_ref[...], w_ref[...],
                             preferred_element_type=jnp.float32,
                             precision=lax.Precision.HIGHEST)

    return pl.pallas_call(
        body,
        out_shape=jax.ShapeDtypeStruct((x.shape[0], w.shape[1]), jnp.float32),
    )(x, w)


def _tc_relu_sum_matmul(p, w):
    def body(p_ref, w_ref, o_ref):
        hidden = jnp.maximum(p_ref[0] + p_ref[1], 0.0)
        o_ref[...] = jnp.dot(hidden, w_ref[...],
                             preferred_element_type=jnp.float32,
                             precision=lax.Precision.HIGHEST)

    return pl.pallas_call(
        body,
        out_shape=jax.ShapeDtypeStruct((p.shape[1], w.shape[1]), jnp.float32),
    )(p, w)


def _tc_sum_partials(q):
    def body(q_ref, o_ref):
        o_ref[...] = q_ref[0] + q_ref[1]

    return pl.pallas_call(
        body,
        out_shape=jax.ShapeDtypeStruct(q.shape[1:], jnp.float32),
    )(q)


def kernel(x, edge_index, W1, W2):
    x = x.astype(jnp.float32)
    src = edge_index[0]
    dst = edge_index[1]

    h = _tc_matmul(x, W1)
    p = _segment_sum_sc(h, src, dst)
    h2 = _tc_relu_sum_matmul(p, W2)
    q = _segment_sum_sc(h2, src, dst)
    return _tc_sum_partials(q)


# speedup vs baseline: 4.8227x; 4.8227x over previous
import functools

import jax
import jax.numpy as jnp
from jax import lax
from jax.experimental import pallas as pl
from jax.experimental.pallas import tpu as pltpu
from jax.experimental.pallas import tpu_sc as plsc

_NC = 2
_NS = 16
_NW = _NC * _NS
_CHUNK = 80
_RB = 80


def _segment_sum_sc(h, src, dst):
    n, d = h.shape
    e = src.shape[0]
    per_w = e // _NW
    n_chunks = per_w // _CHUNK
    n_row_blocks = n // _RB
    mesh = plsc.VectorSubcoreMesh(core_axis_name="c", subcore_axis_name="s")

    @functools.partial(
        pl.kernel,
        out_type=jax.ShapeDtypeStruct((_NC, n, d), jnp.float32),
        mesh=mesh,
        compiler_params=pltpu.CompilerParams(use_tc_tiling_on_sc=False),
        scratch_types=[
            pltpu.VMEM((_CHUNK,), jnp.int32),
            pltpu.VMEM((_CHUNK,), jnp.int32),
            pltpu.VMEM((_CHUNK, d), jnp.float32),
            pltpu.VMEM((_RB, d), jnp.float32),
            pltpu.VMEM_SHARED((n, d), jnp.float32),
            pltpu.SemaphoreType.DMA,
        ],
    )
    def seg_kernel(h_hbm, src_hbm, dst_hbm, out_hbm,
                   src_v, dst_v, rows_v, zeros_v, acc, sem):
        cid = lax.axis_index("c")
        sid = lax.axis_index("s")
        wid = sid * _NC + cid

        zvec = jnp.zeros((16,), jnp.float32)

        @pl.loop(0, _RB)
        def _(r):
            @pl.loop(0, d // 16)
            def _(c):
                zeros_v.at[r, pl.ds(c * 16, 16)][...] = zvec

        @pl.loop(sid, n_row_blocks, step=_NS)
        def _(b):
            pltpu.sync_copy(zeros_v, acc.at[pl.ds(b * _RB, _RB)])

        plsc.subcore_barrier()

        base = wid * per_w

        @pl.loop(0, n_chunks)
        def _(i):
            off = base + i * _CHUNK
            pltpu.sync_copy(src_hbm.at[pl.ds(off, _CHUNK)], src_v)
            pltpu.sync_copy(dst_hbm.at[pl.ds(off, _CHUNK)], dst_v)
            pltpu.async_copy(h_hbm.at[src_v], rows_v, sem).wait()
            pltpu.sync_copy(rows_v, acc.at[dst_v], add=True)

        plsc.subcore_barrier()

        @pl.loop(sid, n_row_blocks, step=_NS)
        def _(b):
            pltpu.sync_copy(acc.at[pl.ds(b * _RB, _RB)],
                            out_hbm.at[cid, pl.ds(b * _RB, _RB)])

    return seg_kernel(h, src, dst)


def _tc_matmul(x, w):
    def body(x_ref, w_ref, o_ref):
        o_ref[...] = jnp.dot(x_ref[...], w_ref[...],
                             preferred_element_type=jnp.float32,
                             precision=lax.Precision.HIGHEST)

    return pl.pallas_call(
        body,
        out_shape=jax.ShapeDtypeStruct((x.shape[0], w.shape[1]), jnp.float32),
    )(x, w)


def _tc_relu_sum_matmul(p, w):
    def body(p_ref, w_ref, o_ref):
        hidden = jnp.maximum(p_ref[0] + p_ref[1], 0.0)
        o_ref[...] = jnp.dot(hidden, w_ref[...],
                             preferred_element_type=jnp.float32,
                             precision=lax.Precision.HIGHEST)

    return pl.pallas_call(
        body,
        out_shape=jax.ShapeDtypeStruct((p.shape[1], w.shape[1]), jnp.float32),
    )(p, w)


def _tc_sum_partials(q):
    def body(q_ref, o_ref):
        o_ref[...] = q_ref[0] + q_ref[1]

    return pl.pallas_call(
        body,
        out_shape=jax.ShapeDtypeStruct(q.shape[1:], jnp.float32),
    )(q)


def kernel(x, edge_index, W1, W2):
    x = x.astype(jnp.float32)
    src = edge_index[0]
    dst = edge_index[1]

    h = _tc_matmul(x, W1)
    p = _segment_sum_sc(h, src, dst)
    h2 = _tc_relu_sum_matmul(p, W2)
    q = _segment_sum_sc(h2, src, dst)
    return _tc_sum_partials(q)


# speedup vs baseline: 13.7093x; 2.8426x over previous
import functools

import jax
import jax.numpy as jnp
from jax import lax
from jax.experimental import pallas as pl
from jax.experimental.pallas import tpu as pltpu
from jax.experimental.pallas import tpu_sc as plsc

_NC = 2
_NS = 16
_NW = _NC * _NS


def _segment_sum_sc(vals, src3, dst3, nbuf):
    n, d = vals.shape
    n_chunks, chunk = src3.shape[1], src3.shape[2]
    assert n_chunks % nbuf == 0
    n_row_blocks = n // chunk
    mesh = plsc.VectorSubcoreMesh(core_axis_name="c", subcore_axis_name="s")

    @functools.partial(
        pl.kernel,
        out_type=jax.ShapeDtypeStruct((_NC, n, d), jnp.float32),
        mesh=mesh,
        compiler_params=pltpu.CompilerParams(use_tc_tiling_on_sc=False),
        scratch_types=[
            pltpu.VMEM((n_chunks, chunk), jnp.int32),
            pltpu.VMEM((nbuf, chunk), jnp.int32),
            pltpu.VMEM((nbuf, chunk, d), jnp.float32),
            pltpu.VMEM_SHARED((n, d), jnp.float32),
        ] + [pltpu.SemaphoreType.DMA] * (2 * nbuf),
    )
    def seg_kernel(vals_hbm, src_hbm, dst_hbm, out_hbm,
                   src_v, dstr_v, rows_v, acc, *sems):
        gsem = sems[:nbuf]
        dsem = sems[nbuf:]
        cid = lax.axis_index("c")
        sid = lax.axis_index("s")
        wid = sid * _NC + cid

        zvec = jnp.zeros((16,), jnp.float32)

        @pl.loop(0, chunk)
        def _(r):
            @pl.loop(0, d // 16)
            def _(c):
                rows_v.at[0, r, pl.ds(c * 16, 16)][...] = zvec

        @pl.loop(sid, n_row_blocks, step=_NS)
        def _(b):
            pltpu.sync_copy(rows_v.at[0], acc.at[pl.ds(b * chunk, chunk)])

        pltpu.sync_copy(src_hbm.at[wid], src_v)

        plsc.subcore_barrier()

        for b in range(nbuf):
            pltpu.async_copy(dst_hbm.at[wid, b], dstr_v.at[b], dsem[b])
            pltpu.async_copy(vals_hbm.at[src_v.at[b]], rows_v.at[b], gsem[b])

        @pl.loop(0, n_chunks - nbuf, step=nbuf)
        def _(i0):
            for b in range(nbuf):
                i = i0 + b
                pltpu.make_async_copy(
                    vals_hbm.at[src_v.at[i]], rows_v.at[b], gsem[b]).wait()
                pltpu.make_async_copy(
                    dst_hbm.at[wid, i], dstr_v.at[b], dsem[b]).wait()
                pltpu.sync_copy(rows_v.at[b], acc.at[dstr_v.at[b]], add=True)
                pltpu.async_copy(
                    dst_hbm.at[wid, i + nbuf], dstr_v.at[b], dsem[b])
                pltpu.async_copy(
                    vals_hbm.at[src_v.at[i + nbuf]], rows_v.at[b], gsem[b])

        for b in range(nbuf):
            i = n_chunks - nbuf + b
            pltpu.make_async_copy(
                vals_hbm.at[src_v.at[i]], rows_v.at[b], gsem[b]).wait()
            pltpu.make_async_copy(
                dst_hbm.at[wid, i], dstr_v.at[b], dsem[b]).wait()
            pltpu.sync_copy(rows_v.at[b], acc.at[dstr_v.at[b]], add=True)

        plsc.subcore_barrier()

        @pl.loop(sid, n_row_blocks, step=_NS)
        def _(b):
            pltpu.sync_copy(acc.at[pl.ds(b * chunk, chunk)],
                            out_hbm.at[cid, pl.ds(b * chunk, chunk)])

    return seg_kernel(vals, src3, dst3)


def _tc_fused_dense(p, w1, w2):
    def body(p_ref, w1_ref, w2_ref, o_ref):
        hidden = jnp.maximum(
            jnp.dot(p_ref[0] + p_ref[1], w1_ref[...],
                    preferred_element_type=jnp.float32,
                    precision=lax.Precision.HIGHEST), 0.0)
        o_ref[...] = jnp.dot(hidden, w2_ref[...],
                             preferred_element_type=jnp.float32,
                             precision=lax.Precision.HIGHEST)

    return pl.pallas_call(
        body,
        out_shape=jax.ShapeDtypeStruct((p.shape[1], w2.shape[1]), jnp.float32),
    )(p, w1, w2)


def _tc_sum_partials(q):
    def body(q_ref, o_ref):
        o_ref[...] = q_ref[0] + q_ref[1]

    return pl.pallas_call(
        body,
        out_shape=jax.ShapeDtypeStruct(q.shape[1:], jnp.float32),
    )(q)


def _edge_slabs(edge_index, chunk):
    e = edge_index.shape[1]
    n_chunks = (e // _NW) // chunk
    src3 = edge_index[0].reshape(_NW, n_chunks, chunk)
    dst3 = edge_index[1].reshape(_NW, n_chunks, chunk)
    return src3, dst3


def kernel(x, edge_index, W1, W2):
    x = x.astype(jnp.float32)
    src40, dst40 = _edge_slabs(edge_index, 40)
    src80, dst80 = _edge_slabs(edge_index, 80)

    p = _segment_sum_sc(x, src40, dst40, nbuf=5)
    h2 = _tc_fused_dense(p, W1, W2)
    q = _segment_sum_sc(h2, src80, dst80, nbuf=5)
    return _tc_sum_partials(q)


# speedup vs baseline: 14.3103x; 1.0438x over previous
import functools

import jax
import jax.numpy as jnp
from jax import lax
from jax.experimental import pallas as pl
from jax.experimental.pallas import tpu as pltpu
from jax.experimental.pallas import tpu_sc as plsc

_NC = 2
_NS = 16
_NW = _NC * _NS


def _segment_sum_sc(vals, src3, dst3, nbuf):
    n, d = vals.shape
    n_chunks, chunk = src3.shape[1], src3.shape[2]
    assert n_chunks % nbuf == 0
    n_row_blocks = n // chunk
    mesh = plsc.VectorSubcoreMesh(core_axis_name="c", subcore_axis_name="s")

    @functools.partial(
        pl.kernel,
        out_type=jax.ShapeDtypeStruct((_NC, n, d), jnp.float32),
        mesh=mesh,
        compiler_params=pltpu.CompilerParams(use_tc_tiling_on_sc=False),
        scratch_types=[
            pltpu.VMEM((n_chunks, chunk), jnp.int32),
            pltpu.VMEM((nbuf, chunk), jnp.int32),
            pltpu.VMEM((nbuf, chunk, d), jnp.float32),
            pltpu.VMEM_SHARED((n, d), jnp.float32),
        ] + [pltpu.SemaphoreType.DMA] * (2 * nbuf),
    )
    def seg_kernel(vals_hbm, src_hbm, dst_hbm, out_hbm,
                   src_v, dstr_v, rows_v, acc, *sems):
        gsem = sems[:nbuf]
        dsem = sems[nbuf:]
        cid = lax.axis_index("c")
        sid = lax.axis_index("s")
        wid = sid * _NC + cid

        pltpu.async_copy(src_hbm.at[wid], src_v, gsem[0])

        zvec = jnp.zeros((16,), jnp.float32)

        @pl.loop(0, chunk)
        def _(r):
            @pl.loop(0, d // 16)
            def _(c):
                rows_v.at[0, r, pl.ds(c * 16, 16)][...] = zvec

        @pl.loop(sid, n_row_blocks, step=_NS)
        def _(b):
            pltpu.async_copy(rows_v.at[0], acc.at[pl.ds(b * chunk, chunk)],
                             dsem[0])

        @pl.loop(sid, n_row_blocks, step=_NS)
        def _(b):
            pltpu.make_async_copy(
                rows_v.at[0], acc.at[pl.ds(b * chunk, chunk)], dsem[0]).wait()

        pltpu.make_async_copy(src_hbm.at[wid], src_v, gsem[0]).wait()

        plsc.subcore_barrier()

        for b in range(nbuf):
            pltpu.async_copy(dst_hbm.at[wid, b], dstr_v.at[b], dsem[b])
            pltpu.async_copy(vals_hbm.at[src_v.at[b]], rows_v.at[b], gsem[b])

        @pl.loop(0, n_chunks - nbuf, step=nbuf)
        def _(i0):
            for b in range(nbuf):
                i = i0 + b
                pltpu.make_async_copy(
                    vals_hbm.at[src_v.at[i]], rows_v.at[b], gsem[b]).wait()
                pltpu.make_async_copy(
                    dst_hbm.at[wid, i], dstr_v.at[b], dsem[b]).wait()
                pltpu.sync_copy(rows_v.at[b], acc.at[dstr_v.at[b]], add=True)
                pltpu.async_copy(
                    dst_hbm.at[wid, i + nbuf], dstr_v.at[b], dsem[b])
                pltpu.async_copy(
                    vals_hbm.at[src_v.at[i + nbuf]], rows_v.at[b], gsem[b])

        for b in range(nbuf):
            i = n_chunks - nbuf + b
            pltpu.make_async_copy(
                vals_hbm.at[src_v.at[i]], rows_v.at[b], gsem[b]).wait()
            pltpu.make_async_copy(
                dst_hbm.at[wid, i], dstr_v.at[b], dsem[b]).wait()
            pltpu.sync_copy(rows_v.at[b], acc.at[dstr_v.at[b]], add=True)

        plsc.subcore_barrier()

        @pl.loop(sid, n_row_blocks, step=_NS)
        def _(b):
            pltpu.async_copy(acc.at[pl.ds(b * chunk, chunk)],
                             out_hbm.at[cid, pl.ds(b * chunk, chunk)],
                             gsem[0])

        @pl.loop(sid, n_row_blocks, step=_NS)
        def _(b):
            pltpu.make_async_copy(
                acc.at[pl.ds(b * chunk, chunk)],
                out_hbm.at[cid, pl.ds(b * chunk, chunk)], gsem[0]).wait()

    return seg_kernel(vals, src3, dst3)


def _tc_fused_dense(p, w1, w2):
    def body(p_ref, w1_ref, w2_ref, o_ref):
        hidden = jnp.maximum(
            jnp.dot(p_ref[0] + p_ref[1], w1_ref[...],
                    preferred_element_type=jnp.float32,
                    precision=lax.Precision.HIGHEST), 0.0)
        o_ref[...] = jnp.dot(hidden, w2_ref[...],
                             preferred_element_type=jnp.float32,
                             precision=lax.Precision.HIGHEST)

    return pl.pallas_call(
        body,
        out_shape=jax.ShapeDtypeStruct((p.shape[1], w2.shape[1]), jnp.float32),
    )(p, w1, w2)


def _tc_sum_partials(q):
    def body(q_ref, o_ref):
        o_ref[...] = q_ref[0] + q_ref[1]

    return pl.pallas_call(
        body,
        out_shape=jax.ShapeDtypeStruct(q.shape[1:], jnp.float32),
    )(q)


def _edge_slabs(edge_index, chunk):
    e = edge_index.shape[1]
    n_chunks = (e // _NW) // chunk
    src3 = edge_index[0].reshape(_NW, n_chunks, chunk)
    dst3 = edge_index[1].reshape(_NW, n_chunks, chunk)
    return src3, dst3


def kernel(x, edge_index, W1, W2):
    x = x.astype(jnp.float32)
    src40, dst40 = _edge_slabs(edge_index, 40)
    src80, dst80 = _edge_slabs(edge_index, 80)

    p = _segment_sum_sc(x, src40, dst40, nbuf=5)
    h2 = _tc_fused_dense(p, W1, W2)
    q = _segment_sum_sc(h2, src80, dst80, nbuf=5)
    return _tc_sum_partials(q)
